# Initial kernel scaffold; baseline (speedup 1.0000x reference)
import jax
import jax.numpy as jnp
from jax.experimental import pallas as pl


def kernel(scores_map, descriptor_map):
    raise NotImplementedError("write your pallas kernel here")


# speedup vs baseline: 1.7581x; 1.7581x over previous
import functools

import jax
import jax.numpy as jnp
from jax import lax
from jax.experimental import pallas as pl
from jax.experimental.pallas import tpu as pltpu
from jax.experimental.pallas import tpu_sc as plsc

RADIUS = 2
TOP_K = 500
KERNEL = 4
H = 512
W = 512
C = 96
NH = H // KERNEL
NW = W // KERNEL
N = NH * NW

SC_TILES = 32
KP_PER_TILE = 16
KP_PAD = SC_TILES * KP_PER_TILE
ELEMS_PER_TILE = KP_PER_TILE * C
IDX_CHUNKS = ELEMS_PER_TILE // 128


def _partner(x, j):
    if j >= 128:
        jr = j // 128
        bit = lax.broadcasted_iota(jnp.int32, (NH, NW), 0) & jr
        return jnp.where(bit == 0, jnp.roll(x, -jr, axis=0), jnp.roll(x, jr, axis=0))
    bit = lax.broadcasted_iota(jnp.int32, (NH, NW), 1) & j
    return jnp.where(bit == 0, jnp.roll(x, -j, axis=1), jnp.roll(x, j, axis=1))


def _detect_body(s_ref, v_ref, x_ref, y_ref):
    s = s_ref[:, :]
    row = lax.broadcasted_iota(jnp.int32, (H, W), 0)
    col = lax.broadcasted_iota(jnp.int32, (H, W), 1)
    good = (row >= RADIUS + 1) & (row < H - RADIUS) & (col >= RADIUS + 1) & (col < W - RADIUS)
    s = jnp.where(good, s, 0.0)

    rv = s
    ra = jnp.zeros((H, W), jnp.int32)
    for d in (1, 2, 3):
        sd = jnp.roll(s, -d, axis=0)
        upd = sd > rv
        rv = jnp.where(upd, sd, rv)
        ra = jnp.where(upd, d, ra)
    cv = rv
    cki = ra
    ckj = jnp.zeros((H, W), jnp.int32)
    for d in (1, 2, 3):
        vd = jnp.roll(rv, -d, axis=1)
        ad = jnp.roll(ra, -d, axis=1)
        upd = (vd > cv) | ((vd == cv) & (ad < cki))
        cv = jnp.where(upd, vd, cv)
        cki = jnp.where(upd, ad, cki)
        ckj = jnp.where(upd, d, ckj)
    gidx = (((row >> 2) * NW + (col >> 2)) << 4) | (cki << 2) | ckj

    sel_l = (lax.broadcasted_iota(jnp.int32, (NH, H), 1)
             == KERNEL * lax.broadcasted_iota(jnp.int32, (NH, H), 0)).astype(jnp.float32)
    sel_r = (lax.broadcasted_iota(jnp.int32, (H, NW), 0)
             == KERNEL * lax.broadcasted_iota(jnp.int32, (H, NW), 1)).astype(jnp.float32)

    def compact_int(a):
        hi = (a >> 15).astype(jnp.float32)
        lo = (a & 0x7FFF).astype(jnp.float32)
        dot = functools.partial(jnp.dot, preferred_element_type=jnp.float32,
                                precision=lax.Precision.HIGHEST)
        hi_c = dot(dot(sel_l, hi), sel_r)
        lo_c = dot(dot(sel_l, lo), sel_r)
        return (hi_c.astype(jnp.int32) << 15) | lo_c.astype(jnp.int32)

    vbits = compact_int(lax.bitcast_convert_type(cv, jnp.int32))
    v = lax.bitcast_convert_type(vbits, jnp.float32)
    ix = compact_int(gidx)

    f = 128 * lax.broadcasted_iota(jnp.int32, (NH, NW), 0) + lax.broadcasted_iota(jnp.int32, (NH, NW), 1)
    k = 2
    while k <= N:
        j = k // 2
        while j >= 1:
            vp = _partner(v, j)
            ixp = _partner(ix, j)
            asc = (f & k) == 0
            lower = (f & j) == 0
            less = (v < vp) | ((v == vp) & (ix < ixp))
            take_min = lower == asc
            sel_self = take_min == less
            v = jnp.where(sel_self, v, vp)
            ix = jnp.where(sel_self, ix, ixp)
            j //= 2
        k *= 2

    v_ref[:, :] = v
    x_ref[:, :] = (((ix >> 4) & (NW - 1)) << 2) | (ix & 3)
    y_ref[:, :] = ((ix >> 11) << 2) | ((ix >> 2) & 3)


def _detect(scores):
    return pl.pallas_call(
        _detect_body,
        out_shape=[
            jax.ShapeDtypeStruct((NH, NW), jnp.float32),
            jax.ShapeDtypeStruct((NH, NW), jnp.int32),
            jax.ShapeDtypeStruct((NH, NW), jnp.int32),
        ],
    )(scores)


def _sc_gather_body(desc_hbm, kpx_hbm, kpy_hbm, out_hbm,
                    kpx_v, kpy_v, idx_v, gat_v, sem):
    wid = lax.axis_index("s") * 2 + lax.axis_index("c")
    base = wid * KP_PER_TILE
    pltpu.sync_copy(kpx_hbm.at[pl.ds(base, KP_PER_TILE)], kpx_v)
    pltpu.sync_copy(kpy_hbm.at[pl.ds(base, KP_PER_TILE)], kpy_v)
    kb = kpy_v[...] * W + kpx_v[...]
    for c in range(C):
        idx_v[c // 8, pl.ds((c % 8) * 16, 16)] = kb + c * (H * W)

    copies = []
    for jc in range(IDX_CHUNKS):
        copies.append(pltpu.async_copy(
            desc_hbm.at[idx_v.at[jc]], gat_v.at[pl.ds(jc * 128, 128)], sem))
    for cp in copies:
        cp.wait()

    acc = jnp.zeros((16,), jnp.float32)
    for c in range(C):
        d = gat_v[pl.ds(c * 16, 16)]
        acc = acc + d * d
    s = 0.5 * (acc + 1.0)
    for _ in range(25):
        s = 0.5 * (s + acc / s)
    r = 1.0 / s
    for c in range(C):
        d = gat_v[pl.ds(c * 16, 16)]
        gat_v[pl.ds(c * 16, 16)] = d * r

    pltpu.sync_copy(gat_v, out_hbm.at[pl.ds(wid * ELEMS_PER_TILE, ELEMS_PER_TILE)])


@functools.lru_cache(maxsize=1)
def _sc_gather():
    return functools.partial(
        pl.kernel,
        mesh=plsc.VectorSubcoreMesh(core_axis_name="c", subcore_axis_name="s"),
        out_type=jax.ShapeDtypeStruct((KP_PAD * C,), jnp.float32),
        scratch_types=[
            pltpu.VMEM((KP_PER_TILE,), jnp.int32),
            pltpu.VMEM((KP_PER_TILE,), jnp.int32),
            pltpu.VMEM((IDX_CHUNKS, 128), jnp.int32),
            pltpu.VMEM((ELEMS_PER_TILE,), jnp.float32),
            pltpu.SemaphoreType.DMA,
        ],
    )(_sc_gather_body)


def kernel(scores_map, descriptor_map):
    s = scores_map.reshape(H, W)
    v_sorted, x_sorted, y_sorted = _detect(s)
    vs = v_sorted.reshape(-1)[-TOP_K:]
    xs = x_sorted.reshape(-1)[-TOP_K:]
    ys = y_sorted.reshape(-1)[-TOP_K:]
    keypoints = jnp.stack([xs, ys], axis=1)

    pad = KP_PAD - TOP_K
    kpx_pad = jnp.concatenate([xs, jnp.zeros((pad,), jnp.int32)])
    kpy_pad = jnp.concatenate([ys, jnp.zeros((pad,), jnp.int32)])
    desc_flat = descriptor_map.reshape(-1)
    out_flat = _sc_gather()(desc_flat, kpx_pad, kpy_pad)
    descriptors = out_flat.reshape(SC_TILES, C, KP_PER_TILE).transpose(0, 2, 1).reshape(KP_PAD, C)[:TOP_K]
    return (keypoints, descriptors, vs)
